# baseline (speedup 1.0000x reference)
import jax
import jax.numpy as jnp
from jax import lax
from jax.experimental import pallas as pl
from jax.experimental.pallas import tpu as pltpu
from jax.experimental.pallas import tpu_sc as plsc

_VOCAB = 1000000
_CHUNK = 16
_NCH = 4
_ROWS = 8388608 // (_NCH * _CHUNK) // 2
_BATCH = 16384

_INFO = plsc.get_sparse_core_info()
_NC = _INFO.num_cores
_NS = _INFO.num_subcores
_NW = _NC * _NS
_NB = _BATCH // _NW
_NI = _NB * _NCH


def _body(x_hbm, t0_hbm, t1_hbm, h0_hbm, h1_hbm, out_hbm,
          x_v, xe_v, r0_v, r1_v, i0_v, i1_v, p0_v, p1_v, s0, s1):
    wid = lax.axis_index("s") * _NC + lax.axis_index("c")
    base = wid * _NB

    pltpu.sync_copy(x_hbm.at[pl.ds(base, _NB)], x_v)

    lane = lax.iota(jnp.int32, 16)
    sub = lane >> 2
    col = lane & 3

    dn = lax.GatherDimensionNumbers(
        offset_dims=(), collapsed_slice_dims=(0,), start_index_map=(0,))

    def shuffle(vec, idx):
        return lax.gather(vec, idx[:, None], dn, slice_sizes=(1,),
                          mode=lax.GatherScatterMode.PROMISE_IN_BOUNDS)

    def expand(i, _):
        xv = x_v[pl.ds(i * 16, 16)]
        for k in range(4):
            rep = shuffle(xv, sub + k * 4)
            xe_v[pl.ds(i * 64 + k * 16, 16)] = rep * 4 + col
        return 0

    lax.fori_loop(0, _NB // 16, expand, 0)

    cp0 = pltpu.async_copy(h0_hbm.at[xe_v], r0_v, s0)
    cp1 = pltpu.async_copy(h1_hbm.at[xe_v], r1_v, s1)
    cp0.wait()
    cp1.wait()

    def repack(i, _):
        i0_v[pl.ds(i * 16, 16)] = r0_v[pl.ds(i * 16, 16)] * 4 + col
        i1_v[pl.ds(i * 16, 16)] = r1_v[pl.ds(i * 16, 16)] * 4 + col
        return 0

    lax.fori_loop(0, _NI // 16, repack, 0)

    cp0 = pltpu.async_copy(t0_hbm.at[i0_v], p0_v, s0)
    cp1 = pltpu.async_copy(t1_hbm.at[i1_v], p1_v, s1)
    cp0.wait()
    cp1.wait()

    def add(i, _):
        for k in range(4):
            j = i * 4 + k
            p0_v[j] = p0_v[j] + p1_v[j]
        return 0

    lax.fori_loop(0, _NI // 4, add, 0)

    pltpu.sync_copy(p0_v, out_hbm.at[pl.ds(base * _NCH, _NI)])


@jax.jit
def _run(x, t0f, t1f, h0f, h1f):
    kern = pl.kernel(
        _body,
        out_type=jax.ShapeDtypeStruct((_BATCH * _NCH, _CHUNK), jnp.float32),
        mesh=plsc.VectorSubcoreMesh(core_axis_name="c", subcore_axis_name="s"),
        compiler_params=pltpu.CompilerParams(use_tc_tiling_on_sc=False),
        scratch_types=[
            pltpu.VMEM((_NB,), jnp.int32),
            pltpu.VMEM((_NI,), jnp.int32),
            pltpu.VMEM((_NI,), jnp.int32),
            pltpu.VMEM((_NI,), jnp.int32),
            pltpu.VMEM((_NI,), jnp.int32),
            pltpu.VMEM((_NI,), jnp.int32),
            pltpu.VMEM((_NI, _CHUNK), jnp.float32),
            pltpu.VMEM((_NI, _CHUNK), jnp.float32),
            pltpu.SemaphoreType.DMA,
            pltpu.SemaphoreType.DMA,
        ],
    )
    return kern(x, t0f, t1f, h0f, h1f)


def kernel(x, table0, table1, h0, h1):
    t0f = table0.reshape(_ROWS * _NCH, _CHUNK)
    t1f = table1.reshape(_ROWS * _NCH, _CHUNK)
    h0f = h0.reshape(_VOCAB * _NCH)
    h1f = h1.reshape(_VOCAB * _NCH)
    out = _run(x, t0f, t1f, h0f, h1f)
    return out.reshape(_BATCH, _NCH * _CHUNK)


# speedup vs baseline: 1.7266x; 1.7266x over previous
import jax
import jax.numpy as jnp
from jax import lax
from jax.experimental import pallas as pl
from jax.experimental.pallas import tpu as pltpu
from jax.experimental.pallas import tpu_sc as plsc

_VOCAB = 1000000
_CHUNK = 16
_NCH = 4
_ROWS = 8388608 // (_NCH * _CHUNK) // 2
_BATCH = 16384

_INFO = plsc.get_sparse_core_info()
_NC = _INFO.num_cores
_NS = _INFO.num_subcores
_NW = _NC * _NS
_NB = _BATCH // _NW
_NI = _NB * _NCH


def _body(x_hbm, t0_hbm, t1_hbm, h0_hbm, h1_hbm, out_hbm,
          x_v, xe_v, r0_v, r1_v, i0_v, i1_v, p0_v, p1_v, s0, s1):
    wid = lax.axis_index("s") * _NC + lax.axis_index("c")
    base = wid * _NB

    pltpu.sync_copy(x_hbm.at[pl.ds(base, _NB)], x_v)

    lane = lax.iota(jnp.int32, 16)
    sub = lane >> 2
    col = lane & 3

    dn = lax.GatherDimensionNumbers(
        offset_dims=(), collapsed_slice_dims=(0,), start_index_map=(0,))

    def shuffle(vec, idx):
        return lax.gather(vec, idx[:, None], dn, slice_sizes=(1,),
                          mode=lax.GatherScatterMode.PROMISE_IN_BOUNDS)

    def expand(i, _):
        xv = x_v[pl.ds(i * 16, 16)]
        for k in range(4):
            rep = shuffle(xv, sub + k * 4)
            xe_v[pl.ds(i * 64 + k * 16, 16)] = rep * 4 + col
        return 0

    lax.fori_loop(0, _NB // 16, expand, 0)

    cp0 = pltpu.async_copy(h0_hbm.at[xe_v], r0_v, s0)
    cp1 = pltpu.async_copy(h1_hbm.at[xe_v], r1_v, s1)
    cp0.wait()
    cp1.wait()

    def repack(i, _):
        g0 = r0_v[pl.ds(i * 16, 16)] & (_ROWS - 1)
        g1 = r1_v[pl.ds(i * 16, 16)] & (_ROWS - 1)
        i0_v[pl.ds(i * 16, 16)] = g0 * 4 + col
        i1_v[pl.ds(i * 16, 16)] = g1 * 4 + col
        return 0

    lax.fori_loop(0, _NI // 16, repack, 0)

    cp0 = pltpu.async_copy(t0_hbm.at[i0_v], p0_v, s0)
    cp1 = pltpu.async_copy(t1_hbm.at[i1_v], p1_v, s1)
    cp0.wait()
    cp1.wait()

    def add(i, _):
        for k in range(4):
            j = i * 4 + k
            p0_v[j] = p0_v[j] + p1_v[j]
        return 0

    lax.fori_loop(0, _NI // 4, add, 0)

    pltpu.sync_copy(p0_v, out_hbm.at[pl.ds(base * _NCH, _NI)])


@jax.jit
def _run(x, table0, table1, h0, h1):
    z = lax.optimization_barrier(jnp.zeros((), jnp.int32))
    h0f = h0.reshape(_VOCAB * _NCH) + z
    h1f = h1.reshape(_VOCAB * _NCH) + z
    t0f = table0.reshape(_ROWS * _NCH, _CHUNK)
    t1f = table1.reshape(_ROWS * _NCH, _CHUNK)

    kern = pl.kernel(
        _body,
        out_type=jax.ShapeDtypeStruct((_BATCH * _NCH, _CHUNK), jnp.float32),
        mesh=plsc.VectorSubcoreMesh(core_axis_name="c", subcore_axis_name="s"),
        compiler_params=pltpu.CompilerParams(use_tc_tiling_on_sc=False),
        scratch_types=[
            pltpu.VMEM((_NB,), jnp.int32),
            pltpu.VMEM((_NI,), jnp.int32),
            pltpu.VMEM((_NI,), jnp.int32),
            pltpu.VMEM((_NI,), jnp.int32),
            pltpu.VMEM((_NI,), jnp.int32),
            pltpu.VMEM((_NI,), jnp.int32),
            pltpu.VMEM((_NI, _CHUNK), jnp.float32),
            pltpu.VMEM((_NI, _CHUNK), jnp.float32),
            pltpu.SemaphoreType.DMA,
            pltpu.SemaphoreType.DMA,
        ],
    )
    out = kern(x, t0f, t1f, h0f, h1f)
    return out.reshape(_BATCH, _NCH * _CHUNK)


def kernel(x, table0, table1, h0, h1):
    return _run(x, table0, table1, h0, h1)


# speedup vs baseline: 10.1355x; 5.8702x over previous
import jax
import jax.numpy as jnp
from jax import lax
from jax.experimental import pallas as pl
from jax.experimental.pallas import tpu as pltpu
from jax.experimental.pallas import tpu_sc as plsc

_VOCAB = 1000000
_CHUNK = 16
_NCH = 4
_ROWS = 8388608 // (_NCH * _CHUNK) // 2
_BATCH = 16384

_INFO = plsc.get_sparse_core_info()
_NC = _INFO.num_cores
_NS = _INFO.num_subcores
_NW = _NC * _NS
_NB = _BATCH // _NW
_NI = _NB * _NCH


def _body(t0_hbm, t1_hbm, r0a_hbm, r1a_hbm, out_hbm,
          r0_v, r1_v, i0_v, i1_v, p0_v, p1_v, s0, s1):
    wid = lax.axis_index("s") * _NC + lax.axis_index("c")
    base = wid * _NI

    pltpu.sync_copy(r0a_hbm.at[pl.ds(base, _NI)], r0_v)
    pltpu.sync_copy(r1a_hbm.at[pl.ds(base, _NI)], r1_v)

    lane = lax.iota(jnp.int32, 16)
    col = lane & 3

    def repack(i, _):
        g0 = r0_v[pl.ds(i * 16, 16)] & (_ROWS - 1)
        g1 = r1_v[pl.ds(i * 16, 16)] & (_ROWS - 1)
        i0_v[pl.ds(i * 16, 16)] = g0 * 4 + col
        i1_v[pl.ds(i * 16, 16)] = g1 * 4 + col
        return 0

    lax.fori_loop(0, _NI // 16, repack, 0)

    cp0 = pltpu.async_copy(t0_hbm.at[i0_v], p0_v, s0)
    cp1 = pltpu.async_copy(t1_hbm.at[i1_v], p1_v, s1)
    cp0.wait()
    cp1.wait()

    def add(i, _):
        for k in range(4):
            j = i * 4 + k
            p0_v[j] = p0_v[j] + p1_v[j]
        return 0

    lax.fori_loop(0, _NI // 4, add, 0)

    pltpu.sync_copy(p0_v, out_hbm.at[pl.ds(base, _NI)])


@jax.jit
def _run(x, table0, table1, h0, h1):
    rows0 = jnp.take(h0, x, axis=0)
    rows1 = jnp.take(h1, x, axis=0)
    z = lax.optimization_barrier(jnp.zeros((), jnp.int32))
    r0f = rows0.reshape(_BATCH * _NCH) + z
    r1f = rows1.reshape(_BATCH * _NCH) + z
    t0f = table0.reshape(_ROWS * _NCH, _CHUNK)
    t1f = table1.reshape(_ROWS * _NCH, _CHUNK)

    kern = pl.kernel(
        _body,
        out_type=jax.ShapeDtypeStruct((_BATCH * _NCH, _CHUNK), jnp.float32),
        mesh=plsc.VectorSubcoreMesh(core_axis_name="c", subcore_axis_name="s"),
        compiler_params=pltpu.CompilerParams(use_tc_tiling_on_sc=False),
        scratch_types=[
            pltpu.VMEM((_NI,), jnp.int32),
            pltpu.VMEM((_NI,), jnp.int32),
            pltpu.VMEM((_NI,), jnp.int32),
            pltpu.VMEM((_NI,), jnp.int32),
            pltpu.VMEM((_NI, _CHUNK), jnp.float32),
            pltpu.VMEM((_NI, _CHUNK), jnp.float32),
            pltpu.SemaphoreType.DMA,
            pltpu.SemaphoreType.DMA,
        ],
    )
    out = kern(t0f, t1f, r0f, r1f)
    return out.reshape(_BATCH, _NCH * _CHUNK)


def kernel(x, table0, table1, h0, h1):
    return _run(x, table0, table1, h0, h1)
